# baseline (speedup 1.0000x reference)
import functools

import jax
import jax.numpy as jnp
from jax.experimental import pallas as pl
from jax.experimental.pallas import tpu as pltpu

START_TOKEN = 1
END_TOKEN = 2
BATCH = 64
BEAM = 8
VOCAB = 100000
LENGTH = 50

_NEG_INF = float("-inf")
_BIG_I32 = 2**30

GW = 4096
NG = 25
NU = GW // 128
CW = NG * 128
TAIL_LO = (NG - 1) * GW
TAIL_W = VOCAB - TAIL_LO
NBB = 4


def _topk_body(cp_ref, pr_ref, en_ref, outs_ref,
               np_ref, voc_ref, beam_ref, eg_ref, og_ref, *scratch):
    a_refs = scratch[:NBB]
    f_refs = scratch[NBB:]

    prms, prs, ens = [], [], []
    for bb in range(NBB):
        pr = pr_ref[bb]
        en = en_ref[bb]
        endedc = en > 0.5
        prm = jnp.where(endedc, _NEG_INF, pr)
        prs.append(pr)
        ens.append(en)
        prms.append(prm)

        for g in range(NG):
            base = g * GW
            nu = NU if g < NG - 1 else TAIL_W // 128 + 1
            acc = cp_ref[bb * BEAM:(bb + 1) * BEAM, base:base + 128]
            uacc = jnp.zeros((BEAM, 128), jnp.int32)
            for u in range(1, nu):
                hi = min(base + 128 * (u + 1), VOCAB)
                w = hi - (base + 128 * u)
                x = cp_ref[bb * BEAM:(bb + 1) * BEAM, base + 128 * u: hi]
                if w < 128:
                    x = jnp.concatenate(
                        [x, jnp.full((BEAM, 128 - w), _NEG_INF, jnp.float32)],
                        axis=1)
                upd = x > acc
                uacc = jnp.where(upd, u, uacc)
                acc = jnp.maximum(acc, x)
            a_refs[bb][:, g * 128:(g + 1) * 128] = acc
            f_refs[bb][:, g * 128:(g + 1) * 128] = uacc

        rowc = jax.lax.broadcasted_iota(jnp.int32, (BEAM, CW), 0)
        colc = jax.lax.broadcasted_iota(jnp.int32, (BEAM, CW), 1)
        ap = a_refs[bb][...] + prm
        endfix = endedc & (colc == END_TOKEN)
        ap = jnp.where(endfix, pr, ap)
        u0 = jnp.where(endfix, 0, f_refs[bb][...])
        flat0 = rowc * VOCAB + (colc // 128) * GW + (colc % 128) + u0 * 128
        a_refs[bb][...] = ap
        f_refs[bb][...] = flat0

    row8 = jax.lax.broadcasted_iota(jnp.int32, (BEAM, 1), 0)
    lane_g = jax.lax.broadcasted_iota(jnp.int32, (BEAM, GW), 1)
    lanemod_g = lane_g % 128
    lane_t = jax.lax.broadcasted_iota(jnp.int32, (BEAM, TAIL_W), 1)
    lanemod_t = lane_t % 128
    lane8 = jax.lax.broadcasted_iota(jnp.int32, (1, BEAM), 1)

    mvecs = [jnp.zeros((1, BEAM), jnp.float32) for _ in range(NBB)]
    fvecs = [jnp.zeros((1, BEAM), jnp.int32) for _ in range(NBB)]

    for r in range(BEAM):
        st = [dict() for _ in range(NBB)]
        for bb in range(NBB):
            st[bb]["A"] = a_refs[bb][...]
            st[bb]["F"] = f_refs[bb][...]
        for bb in range(NBB):
            st[bb]["mx"] = jnp.max(st[bb]["A"], axis=(0, 1), keepdims=True)
        for bb in range(NBB):
            d = st[bb]
            d["fm"] = jnp.min(jnp.where(d["A"] == d["mx"], d["F"], _BIG_I32),
                              axis=(0, 1), keepdims=True)
        for bb in range(NBB):
            d = st[bb]
            d["kb"] = d["fm"] // VOCAB
            d["vb"] = d["fm"] % VOCAB
            d["gb"] = d["vb"] // GW
            d["lb"] = d["vb"] % 128
            d["istail"] = d["gb"] == NG - 1
            g_s = jnp.max(d["gb"])
            d["s"] = jnp.minimum(g_s, NG - 2) * GW
            d["rowm"] = row8 == d["kb"]
        for bb in range(NBB):
            d = st[bb]
            Xd = cp_ref[bb * BEAM:(bb + 1) * BEAM, pl.ds(d["s"], GW)] + prms[bb]
            keep_d = (d["rowm"] & jnp.logical_not(d["istail"])
                      & (lanemod_g == d["lb"])
                      & ((Xd < d["mx"])
                         | ((Xd == d["mx"]) & (lane_g > d["vb"] - d["s"]))))
            d["Xmd"] = jnp.where(keep_d, Xd, _NEG_INF)
            Xt = cp_ref[bb * BEAM:(bb + 1) * BEAM, TAIL_LO:VOCAB] + prms[bb]
            keep_t = (d["rowm"] & d["istail"]
                      & (lanemod_t == d["lb"])
                      & ((Xt < d["mx"])
                         | ((Xt == d["mx"]) & (lane_t > d["vb"] - TAIL_LO))))
            d["Xmt"] = jnp.where(keep_t, Xt, _NEG_INF)
        for bb in range(NBB):
            d = st[bb]
            d["nb_d"] = jnp.max(d["Xmd"], axis=(0, 1), keepdims=True)
            d["nb_t"] = jnp.max(d["Xmt"], axis=(0, 1), keepdims=True)
        for bb in range(NBB):
            d = st[bb]
            d["nv_d"] = jnp.min(
                jnp.where(d["Xmd"] == d["nb_d"], lane_g + d["s"], _BIG_I32),
                axis=(0, 1), keepdims=True)
            d["nv_t"] = jnp.min(
                jnp.where(d["Xmt"] == d["nb_t"], lane_t + TAIL_LO, _BIG_I32),
                axis=(0, 1), keepdims=True)
        for bb in range(NBB):
            d = st[bb]
            nb = jnp.maximum(d["nb_d"], d["nb_t"])
            nv = jnp.where(d["nb_t"] > d["nb_d"], d["nv_t"], d["nv_d"])
            nf = d["kb"] * VOCAB + nv
            popm = d["F"] == d["fm"]
            a_refs[bb][...] = jnp.where(popm, nb, d["A"])
            f_refs[bb][...] = jnp.where(popm, nf, d["F"])
            mvecs[bb] = jnp.where(
                lane8 == r, jnp.broadcast_to(d["mx"], (1, BEAM)), mvecs[bb])
            fvecs[bb] = jnp.where(
                lane8 == r, jnp.broadcast_to(d["fm"], (1, BEAM)), fvecs[bb])

    for bb in range(NBB):
        vvec = fvecs[bb] % VOCAB
        bvec = fvecs[bb] // VOCAB

        rows88 = jax.lax.broadcasted_iota(jnp.int32, (BEAM, BEAM), 0)
        oh = (rows88 == jnp.broadcast_to(bvec, (BEAM, BEAM))).astype(
            jnp.float32)

        outs_f = outs_ref[bb].astype(jnp.float32)
        og = jax.lax.dot(outs_f, oh, precision=jax.lax.Precision.HIGHEST,
                         preferred_element_type=jnp.float32)
        og_ref[bb, :LENGTH, :] = og.astype(jnp.int32)
        og_ref[bb, LENGTH:, :] = vvec

        en_b = jnp.broadcast_to(ens[bb], (BEAM, BEAM))
        eg = jnp.sum(en_b * oh, axis=0, keepdims=True)
        ended_new = jnp.where(vvec == END_TOKEN, 1.0, eg)

        np_ref[bb] = mvecs[bb]
        voc_ref[bb] = vvec
        beam_ref[bb] = bvec
        eg_ref[bb] = ended_new


def _half(cp2, pr3, en3, outs_t, nb):
    grid = (nb // NBB,)
    out_shapes = (
        jax.ShapeDtypeStruct((nb, 1, BEAM), jnp.float32),
        jax.ShapeDtypeStruct((nb, 1, BEAM), jnp.int32),
        jax.ShapeDtypeStruct((nb, 1, BEAM), jnp.int32),
        jax.ShapeDtypeStruct((nb, 1, BEAM), jnp.float32),
        jax.ShapeDtypeStruct((nb, LENGTH + 1, BEAM), jnp.int32),
    )
    return pl.pallas_call(
        _topk_body,
        grid=grid,
        in_specs=[
            pl.BlockSpec((NBB * BEAM, VOCAB), lambda b: (b, 0)),
            pl.BlockSpec((NBB, BEAM, 1), lambda b: (b, 0, 0)),
            pl.BlockSpec((NBB, BEAM, 1), lambda b: (b, 0, 0)),
            pl.BlockSpec((NBB, LENGTH, BEAM), lambda b: (b, 0, 0)),
        ],
        out_specs=(
            pl.BlockSpec((NBB, 1, BEAM), lambda b: (b, 0, 0)),
            pl.BlockSpec((NBB, 1, BEAM), lambda b: (b, 0, 0)),
            pl.BlockSpec((NBB, 1, BEAM), lambda b: (b, 0, 0)),
            pl.BlockSpec((NBB, 1, BEAM), lambda b: (b, 0, 0)),
            pl.BlockSpec((NBB, LENGTH + 1, BEAM), lambda b: (b, 0, 0)),
        ),
        out_shape=out_shapes,
        scratch_shapes=(
            [pltpu.VMEM((BEAM, CW), jnp.float32) for _ in range(NBB)]
            + [pltpu.VMEM((BEAM, CW), jnp.int32) for _ in range(NBB)]
        ),
    )(cp2, pr3, en3, outs_t)


@functools.partial(jax.jit, static_argnums=())
def kernel(cur_proba, proba, outs, is_ended):
    pr3 = proba.reshape(BATCH, BEAM, 1)
    en3 = is_ended.astype(jnp.float32).reshape(BATCH, BEAM, 1)
    outs_t = outs.transpose(1, 0, 2)

    hb = BATCH // 2
    outs_list = []
    for h in range(2):
        cp2 = cur_proba[h * hb * BEAM:(h + 1) * hb * BEAM].reshape(
            hb * BEAM, VOCAB)
        outs_list.append(_half(
            cp2, pr3[h * hb:(h + 1) * hb], en3[h * hb:(h + 1) * hb],
            outs_t[h * hb:(h + 1) * hb], hb))
    np_o, voc_o, beam_o, eg_o, og_o = (
        jnp.concatenate([a, b], axis=0) for a, b in zip(*outs_list))

    new_proba = np_o.reshape(BATCH, BEAM)
    topk_voc = voc_o.reshape(BATCH, BEAM)
    topk_beam = beam_o.reshape(BATCH, BEAM)
    is_ended_new = eg_o.reshape(BATCH, BEAM) > 0.5
    outs_new = og_o.transpose(1, 0, 2)
    cur_input = topk_voc.reshape(BATCH * BEAM, 1)
    return (cur_input, new_proba, outs_new, is_ended_new, topk_beam)


# speedup vs baseline: 1.4769x; 1.4769x over previous
import functools

import jax
import jax.numpy as jnp
from jax.experimental import pallas as pl
from jax.experimental.pallas import tpu as pltpu

START_TOKEN = 1
END_TOKEN = 2
BATCH = 64
BEAM = 8
VOCAB = 100000
LENGTH = 50

_NEG_INF = float("-inf")
_BIG_I32 = 2**30

GW = 4096
NG = 25
NU = GW // 128
CW = NG * 128
TAIL_LO = (NG - 1) * GW
TAIL_W = VOCAB - TAIL_LO
NBB = 4


def _topk_body(cp_ref, pr_ref, en_ref, outs_ref,
               np_ref, voc_ref, beam_ref, eg_ref, og_ref, *scratch):
    a_refs = scratch[:NBB]
    f_refs = scratch[NBB:]

    prms, prs, ens = [], [], []
    for bb in range(NBB):
        pr = pr_ref[bb]
        en = en_ref[bb]
        endedc = en > 0.5
        prm = jnp.where(endedc, _NEG_INF, pr)
        prs.append(pr)
        ens.append(en)
        prms.append(prm)

        for g in range(NG):
            base = g * GW
            nu = NU if g < NG - 1 else TAIL_W // 128 + 1
            acc = cp_ref[bb * BEAM:(bb + 1) * BEAM, base:base + 128]
            uacc = jnp.zeros((BEAM, 128), jnp.int32)
            for u in range(1, nu):
                hi = min(base + 128 * (u + 1), VOCAB)
                w = hi - (base + 128 * u)
                x = cp_ref[bb * BEAM:(bb + 1) * BEAM, base + 128 * u: hi]
                if w < 128:
                    x = jnp.concatenate(
                        [x, jnp.full((BEAM, 128 - w), _NEG_INF, jnp.float32)],
                        axis=1)
                upd = x > acc
                uacc = jnp.where(upd, u, uacc)
                acc = jnp.maximum(acc, x)
            a_refs[bb][:, g * 128:(g + 1) * 128] = acc
            f_refs[bb][:, g * 128:(g + 1) * 128] = uacc

        rowc = jax.lax.broadcasted_iota(jnp.int32, (BEAM, CW), 0)
        colc = jax.lax.broadcasted_iota(jnp.int32, (BEAM, CW), 1)
        ap = a_refs[bb][...] + prm
        endfix = endedc & (colc == END_TOKEN)
        ap = jnp.where(endfix, pr, ap)
        u0 = jnp.where(endfix, 0, f_refs[bb][...])
        flat0 = rowc * VOCAB + (colc // 128) * GW + (colc % 128) + u0 * 128
        a_refs[bb][...] = ap
        f_refs[bb][...] = flat0

    row8 = jax.lax.broadcasted_iota(jnp.int32, (BEAM, 1), 0)
    lane_g = jax.lax.broadcasted_iota(jnp.int32, (BEAM, GW), 1)
    lanemod_g = lane_g % 128
    lane_t = jax.lax.broadcasted_iota(jnp.int32, (BEAM, TAIL_W), 1)
    lanemod_t = lane_t % 128
    lane8 = jax.lax.broadcasted_iota(jnp.int32, (1, BEAM), 1)

    mvecs = [jnp.zeros((1, BEAM), jnp.float32) for _ in range(NBB)]
    fvecs = [jnp.zeros((1, BEAM), jnp.int32) for _ in range(NBB)]

    for r in range(BEAM):
        st = [dict() for _ in range(NBB)]
        for bb in range(NBB):
            st[bb]["A"] = a_refs[bb][...]
            st[bb]["F"] = f_refs[bb][...]
        for bb in range(NBB):
            st[bb]["mx"] = jnp.max(st[bb]["A"], axis=(0, 1), keepdims=True)
        for bb in range(NBB):
            d = st[bb]
            d["fm"] = jnp.min(jnp.where(d["A"] == d["mx"], d["F"], _BIG_I32),
                              axis=(0, 1), keepdims=True)
        for bb in range(NBB):
            d = st[bb]
            d["kb"] = d["fm"] // VOCAB
            d["vb"] = d["fm"] % VOCAB
            d["gb"] = d["vb"] // GW
            d["lb"] = d["vb"] % 128
            d["istail"] = d["gb"] == NG - 1
            g_s = jnp.max(d["gb"])
            d["s"] = jnp.minimum(g_s, NG - 2) * GW
            d["rowm"] = row8 == d["kb"]
        for bb in range(NBB):
            d = st[bb]
            Xd = cp_ref[bb * BEAM:(bb + 1) * BEAM, pl.ds(d["s"], GW)] + prms[bb]
            keep_d = (d["rowm"] & jnp.logical_not(d["istail"])
                      & (lanemod_g == d["lb"])
                      & ((Xd < d["mx"])
                         | ((Xd == d["mx"]) & (lane_g > d["vb"] - d["s"]))))
            d["Xmd"] = jnp.where(keep_d, Xd, _NEG_INF)
            Xt = cp_ref[bb * BEAM:(bb + 1) * BEAM, TAIL_LO:VOCAB] + prms[bb]
            keep_t = (d["rowm"] & d["istail"]
                      & (lanemod_t == d["lb"])
                      & ((Xt < d["mx"])
                         | ((Xt == d["mx"]) & (lane_t > d["vb"] - TAIL_LO))))
            d["Xmt"] = jnp.where(keep_t, Xt, _NEG_INF)
        for bb in range(NBB):
            d = st[bb]
            d["nb_d"] = jnp.max(d["Xmd"], axis=(0, 1), keepdims=True)
            d["nb_t"] = jnp.max(d["Xmt"], axis=(0, 1), keepdims=True)
        for bb in range(NBB):
            d = st[bb]
            d["nv_d"] = jnp.min(
                jnp.where(d["Xmd"] == d["nb_d"], lane_g + d["s"], _BIG_I32),
                axis=(0, 1), keepdims=True)
            d["nv_t"] = jnp.min(
                jnp.where(d["Xmt"] == d["nb_t"], lane_t + TAIL_LO, _BIG_I32),
                axis=(0, 1), keepdims=True)
        for bb in range(NBB):
            d = st[bb]
            nb = jnp.maximum(d["nb_d"], d["nb_t"])
            nv = jnp.where(d["nb_t"] > d["nb_d"], d["nv_t"], d["nv_d"])
            nf = d["kb"] * VOCAB + nv
            popm = d["F"] == d["fm"]
            a_refs[bb][...] = jnp.where(popm, nb, d["A"])
            f_refs[bb][...] = jnp.where(popm, nf, d["F"])
            mvecs[bb] = jnp.where(
                lane8 == r, jnp.broadcast_to(d["mx"], (1, BEAM)), mvecs[bb])
            fvecs[bb] = jnp.where(
                lane8 == r, jnp.broadcast_to(d["fm"], (1, BEAM)), fvecs[bb])

    for bb in range(NBB):
        vvec = fvecs[bb] % VOCAB
        bvec = fvecs[bb] // VOCAB

        rows88 = jax.lax.broadcasted_iota(jnp.int32, (BEAM, BEAM), 0)
        oh = (rows88 == jnp.broadcast_to(bvec, (BEAM, BEAM))).astype(
            jnp.float32)

        outs_f = outs_ref[bb].astype(jnp.float32)
        og = jax.lax.dot(outs_f, oh, precision=jax.lax.Precision.HIGHEST,
                         preferred_element_type=jnp.float32)
        og_ref[bb, :LENGTH, :] = og.astype(jnp.int32)
        og_ref[bb, LENGTH:, :] = vvec

        en_b = jnp.broadcast_to(ens[bb], (BEAM, BEAM))
        eg = jnp.sum(en_b * oh, axis=0, keepdims=True)
        ended_new = jnp.where(vvec == END_TOKEN, 1.0, eg)

        np_ref[bb] = mvecs[bb]
        voc_ref[bb] = vvec
        beam_ref[bb] = bvec
        eg_ref[bb] = ended_new


@functools.partial(jax.jit, static_argnums=())
def kernel(cur_proba, proba, outs, is_ended):
    cp2 = cur_proba.reshape(BATCH * BEAM, VOCAB)
    pr3 = proba.reshape(BATCH, BEAM, 1)
    en3 = is_ended.astype(jnp.float32).reshape(BATCH, BEAM, 1)
    outs_t = outs.transpose(1, 0, 2)

    grid = (BATCH // NBB,)
    out_shapes = (
        jax.ShapeDtypeStruct((BATCH, 1, BEAM), jnp.float32),
        jax.ShapeDtypeStruct((BATCH, 1, BEAM), jnp.int32),
        jax.ShapeDtypeStruct((BATCH, 1, BEAM), jnp.int32),
        jax.ShapeDtypeStruct((BATCH, 1, BEAM), jnp.float32),
        jax.ShapeDtypeStruct((BATCH, LENGTH + 1, BEAM), jnp.int32),
    )
    np_o, voc_o, beam_o, eg_o, og_o = pl.pallas_call(
        _topk_body,
        grid=grid,
        in_specs=[
            pl.BlockSpec((NBB * BEAM, VOCAB), lambda b: (b, 0)),
            pl.BlockSpec((NBB, BEAM, 1), lambda b: (b, 0, 0)),
            pl.BlockSpec((NBB, BEAM, 1), lambda b: (b, 0, 0)),
            pl.BlockSpec((NBB, LENGTH, BEAM), lambda b: (b, 0, 0)),
        ],
        out_specs=(
            pl.BlockSpec((NBB, 1, BEAM), lambda b: (b, 0, 0)),
            pl.BlockSpec((NBB, 1, BEAM), lambda b: (b, 0, 0)),
            pl.BlockSpec((NBB, 1, BEAM), lambda b: (b, 0, 0)),
            pl.BlockSpec((NBB, 1, BEAM), lambda b: (b, 0, 0)),
            pl.BlockSpec((NBB, LENGTH + 1, BEAM), lambda b: (b, 0, 0)),
        ),
        out_shape=out_shapes,
        scratch_shapes=(
            [pltpu.VMEM((BEAM, CW), jnp.float32) for _ in range(NBB)]
            + [pltpu.VMEM((BEAM, CW), jnp.int32) for _ in range(NBB)]
        ),
    )(cp2, pr3, en3, outs_t)

    new_proba = np_o.reshape(BATCH, BEAM)
    topk_voc = voc_o.reshape(BATCH, BEAM)
    topk_beam = beam_o.reshape(BATCH, BEAM)
    is_ended_new = eg_o.reshape(BATCH, BEAM) > 0.5
    outs_new = og_o.transpose(1, 0, 2)
    cur_input = topk_voc.reshape(BATCH * BEAM, 1)
    return (cur_input, new_proba, outs_new, is_ended_new, topk_beam)
